# baseline (speedup 1.0000x reference)
import functools

import jax
import jax.numpy as jnp
from jax import lax
from jax.experimental import pallas as pl
from jax.experimental.pallas import tpu as pltpu
from jax.experimental.pallas import tpu_sc as plsc

L = 16
NWORK = 32
W = 50000
K = 4096
VPB = 25
NACC = 4
NEG_BIG = -3.0e38


def _select_lane(vec, idx):
    lane = lax.iota(jnp.int32, L)
    return jnp.sum(jnp.where(lane == idx, vec, jnp.zeros_like(vec)))


def _moments_pass(buf, j0, tr, carry, zk_ref):
    in_rng = jnp.logical_and(tr >= j0, tr < j0 + W)
    tloc = jnp.clip(tr - j0, 0, W - 1)

    @pl.when(in_rng)
    def _():
        wbase = pl.multiple_of((tloc // L) * L, L)
        zk_ref[0] = _select_lane(buf[pl.ds(wbase, L)], tloc % L)

    def step(bi, c):
        rm, s1, s2 = c
        base = bi * (VPB * L)
        s1 = list(s1)
        s2 = list(s2)
        bms = []
        for u in range(VPB):
            v = buf[pl.ds(base + u * L, L)]
            s1[u % NACC] = s1[u % NACC] + v
            s2[u % NACC] = s2[u % NACC] + v * v
            if u % NACC == 0:
                bms.append(v)
            else:
                bms[-1] = jnp.maximum(bms[-1], v)
        for b in bms:
            rm = jnp.maximum(rm, b)
        return rm, tuple(s1), tuple(s2)

    return lax.fori_loop(0, W // (VPB * L), step, carry, unroll=False)


def _filter_pass(buf, thrv, cand, offv):
    onev = jnp.full((L,), 1, jnp.int32)
    zerov = jnp.zeros((L,), jnp.int32)
    kcap = jnp.full((L,), K - 1, jnp.int32)

    def step(bi, offv):
        base = bi * (VPB * L)
        for u in range(VPB):
            v = buf[pl.ds(base + u * L, L)]
            msk = v >= thrv
            pos = offv + plsc.cumsum(jnp.where(msk, onev, zerov)) - 1
            plsc.store_scatter(cand, [jnp.minimum(pos, kcap)], v, mask=msk)
            offv = offv + plsc.all_reduce_population_count(msk)
        return offv

    return lax.fori_loop(0, W // (VPB * L), step, offv, unroll=False)


def _sum_count_above(cand, n, t):
    nv = (n + L - 1) // L
    lane = lax.iota(jnp.int32, L)

    def body(i, c):
        sv, cv = c
        v = cand[pl.ds(i * L, L)]
        valid = (i * L + lane) < n
        msk = jnp.logical_and(v > t, valid)
        sv = sv + jnp.where(msk, v, 0.0)
        cv = cv + jnp.where(msk, 1.0, 0.0)
        return sv, cv

    z = jnp.zeros((L,), jnp.float32)
    sv, cv = lax.fori_loop(0, nv, body, (z, z), unroll=False)
    return jnp.sum(sv), jnp.sum(cv)


def _sparsemax_loss_sc(inp, target, *, b_per_w, ncols):
    mesh = plsc.VectorSubcoreMesh(core_axis_name="c", subcore_axis_name="s")

    @functools.partial(
        pl.kernel,
        out_type=jax.ShapeDtypeStruct((inp.shape[0] // ncols,), jnp.float32),
        mesh=mesh,
        compiler_params=pltpu.CompilerParams(needs_layout_passes=False),
        scratch_types=[
            pltpu.VMEM((W,), jnp.float32),
            pltpu.VMEM((W,), jnp.float32),
            pltpu.VMEM((K,), jnp.float32),
            pltpu.VMEM((b_per_w,), jnp.int32),
            pltpu.VMEM((b_per_w,), jnp.float32),
            pltpu.SMEM((1,), jnp.int32),
            pltpu.SMEM((1,), jnp.float32),
            pltpu.SemaphoreType.DMA,
            pltpu.SemaphoreType.DMA,
        ],
    )
    def k(inp_hbm, tgt_hbm, out_hbm, buf0, buf1, cand, tgt_v, loss_v,
          off_ref, zk_ref, sem0, sem1):
        wid = lax.axis_index("s") * 2 + lax.axis_index("c")
        base = wid * b_per_w
        pltpu.sync_copy(tgt_hbm.at[pl.ds(base, b_per_w)], tgt_v)

        pltpu.async_copy(inp_hbm.at[pl.ds(base * ncols, W)], buf0, sem0)
        pltpu.async_copy(inp_hbm.at[pl.ds(base * ncols + W, W)], buf1, sem1)

        def row_body(rl, laccs):
            r = base + rl
            trf = jnp.float32(0.0)
            for w in range(b_per_w // L):
                tw = tgt_v[pl.ds(w * L, L)].astype(jnp.float32)
                trf = trf + jnp.sum(
                    jnp.where(lax.iota(jnp.int32, L) == rl - w * L, tw,
                              jnp.zeros_like(tw)))
            tr = trf.astype(jnp.int32)
            zk_ref[0] = 0.0

            negv = jnp.full((L,), NEG_BIG, jnp.float32)
            zv = jnp.zeros((L,), jnp.float32)
            carry = (negv, (zv,) * NACC, (zv,) * NACC)

            pltpu.make_async_copy(
                inp_hbm.at[pl.ds(r * ncols, W)], buf0, sem0).wait()
            carry = _moments_pass(buf0, 0, tr, carry, zk_ref)
            pltpu.make_async_copy(
                inp_hbm.at[pl.ds(r * ncols + W, W)], buf1, sem1).wait()
            rm, s1t, s2t = _moments_pass(buf1, W, tr, carry, zk_ref)

            m = jnp.max(rm)
            s1 = jnp.sum(s1t[0] + s1t[1] + s1t[2] + s1t[3])
            s2 = jnp.sum(s2t[0] + s2t[1] + s2t[2] + s2t[3])
            thrv = jnp.full((L,), m - 1.0, jnp.float32)

            offv = _filter_pass(buf0, thrv, cand, jnp.zeros((L,), jnp.int32))

            @pl.when(rl < b_per_w - 1)
            def _():
                pltpu.async_copy(
                    inp_hbm.at[pl.ds((r + 1) * ncols, W)], buf0, sem0)

            offv = _filter_pass(buf1, thrv, cand, offv)

            @pl.when(rl < b_per_w - 1)
            def _():
                pltpu.async_copy(
                    inp_hbm.at[pl.ds((r + 1) * ncols + W, W)], buf1, sem1)

            n = jnp.minimum(offv[0], K)

            def mcond(c):
                t_old, t_new, it = c
                return jnp.logical_and(t_old != t_new, it < 32)

            def mbody(c):
                _, t, it = c
                s, cnt = _sum_count_above(cand, n, t)
                tv = jnp.full((L,), s - 1.0, jnp.float32) / jnp.full(
                    (L,), cnt, jnp.float32)
                return t, tv[0], it + 1

            _, tau_x, _ = lax.while_loop(
                mcond, mbody, (jnp.float32(1.0), jnp.float32(NEG_BIG),
                               jnp.int32(0)))

            lane = lax.iota(jnp.int32, L)

            def corr_body(i, c):
                sv, cv = c
                v = cand[pl.ds(i * L, L)]
                valid = (i * L + lane) < n
                msk = jnp.logical_and(v > tau_x, valid)
                z = v - m
                sv = sv + jnp.where(msk, z * z, 0.0)
                cv = cv + jnp.where(msk, 1.0, 0.0)
                return sv, cv

            sv, cv = lax.fori_loop(0, (n + L - 1) // L, corr_body, (zv, zv),
                                   unroll=False)
            corr = jnp.sum(sv)
            cnt = jnp.sum(cv)

            t2 = s2 - 2.0 * m * s1 + jnp.float32(ncols) * m * m
            tau_z = tau_x - m
            s2t_ = t2 - corr - tau_z * tau_z * (jnp.float32(ncols) - cnt)
            lossval = 0.5 * (s2t_ + 1.0) - zk_ref[0]
            return tuple(
                laccs[w] + jnp.where(lane == rl - w * L,
                                     jnp.full((L,), lossval, jnp.float32),
                                     jnp.zeros((L,), jnp.float32))
                for w in range(b_per_w // L))

        laccs = lax.fori_loop(
            0, b_per_w, row_body,
            tuple(jnp.zeros((L,), jnp.float32) for _ in range(b_per_w // L)),
            unroll=False)
        for w in range(b_per_w // L):
            loss_v[pl.ds(w * L, L)] = laccs[w]
        pltpu.sync_copy(loss_v, out_hbm.at[pl.ds(base, b_per_w)])

    return k(inp, target)


def kernel(input, target):
    B, C = input.shape
    assert B % NWORK == 0 and B // NWORK % L == 0
    assert C == 2 * W and W % (VPB * L) == 0 and W % 8 == 0
    losses = _sparsemax_loss_sc(
        input.reshape(-1), target.astype(jnp.int32), b_per_w=B // NWORK,
        ncols=C)
    return jnp.mean(losses)


# speedup vs baseline: 1.6612x; 1.6612x over previous
import functools

import jax
import jax.numpy as jnp
from jax import lax
from jax.experimental import pallas as pl
from jax.experimental.pallas import tpu as pltpu
from jax.experimental.pallas import tpu_sc as plsc

L = 16
NWORK = 32
W = 50000
K = 16384
VPB = 25
NACC = 4
NEG_BIG = -3.0e38


def _select_lane(vec, idx):
    lane = lax.iota(jnp.int32, L)
    return jnp.sum(jnp.where(lane == idx, vec, jnp.zeros_like(vec)))


def _moments_pass(buf, j0, tr, carry, zk_ref):
    in_rng = jnp.logical_and(tr >= j0, tr < j0 + W)
    tloc = jnp.clip(tr - j0, 0, W - 1)

    @pl.when(in_rng)
    def _():
        wbase = pl.multiple_of((tloc // L) * L, L)
        zk_ref[0] = _select_lane(buf[pl.ds(wbase, L)], tloc % L)

    def step(bi, c):
        rm, s1, s2 = c
        base = bi * (VPB * L)
        s1 = list(s1)
        s2 = list(s2)
        bms = []
        for u in range(VPB):
            v = buf[pl.ds(base + u * L, L)]
            s1[u % NACC] = s1[u % NACC] + v
            s2[u % NACC] = s2[u % NACC] + v * v
            if u % NACC == 0:
                bms.append(v)
            else:
                bms[-1] = jnp.maximum(bms[-1], v)
        for b in bms:
            rm = jnp.maximum(rm, b)
        return rm, tuple(s1), tuple(s2)

    return lax.fori_loop(0, W // (VPB * L), step, carry, unroll=False)


def _filter_pass(buf, thrv, cand, offv):
    lanevec = lax.iota(jnp.int32, L)
    kcap = jnp.full((L,), K - L, jnp.int32)
    stepv = jnp.full((L,), L, jnp.int32)
    zerov = jnp.zeros((L,), jnp.int32)

    def step(bi, offv):
        base = bi * (VPB * L)
        for u in range(VPB):
            v = buf[pl.ds(base + u * L, L)]
            pcnt = plsc.all_reduce_population_count(v >= thrv)
            offc = jnp.minimum(offv, kcap)
            plsc.store_scatter(cand, [offc + lanevec], v)
            offv = offc + jnp.where(pcnt >= 1, stepv, zerov)
        return offv

    return lax.fori_loop(0, W // (VPB * L), step, offv, unroll=False)


def _sum_count_above(cand, n, t):
    nv = (n + L - 1) // L
    lane = lax.iota(jnp.int32, L)

    def body(i, c):
        sv, cv = c
        v = cand[pl.ds(i * L, L)]
        valid = (i * L + lane) < n
        msk = jnp.logical_and(v > t, valid)
        sv = sv + jnp.where(msk, v, 0.0)
        cv = cv + jnp.where(msk, 1.0, 0.0)
        return sv, cv

    z = jnp.zeros((L,), jnp.float32)
    sv, cv = lax.fori_loop(0, nv, body, (z, z), unroll=False)
    return jnp.sum(sv), jnp.sum(cv)


def _sparsemax_loss_sc(inp, target, *, b_per_w, ncols):
    mesh = plsc.VectorSubcoreMesh(core_axis_name="c", subcore_axis_name="s")

    @functools.partial(
        pl.kernel,
        out_type=jax.ShapeDtypeStruct((inp.shape[0] // ncols,), jnp.float32),
        mesh=mesh,
        compiler_params=pltpu.CompilerParams(needs_layout_passes=False),
        scratch_types=[
            pltpu.VMEM((W,), jnp.float32),
            pltpu.VMEM((W,), jnp.float32),
            pltpu.VMEM((K,), jnp.float32),
            pltpu.VMEM((b_per_w,), jnp.int32),
            pltpu.VMEM((b_per_w,), jnp.float32),
            pltpu.SMEM((1,), jnp.int32),
            pltpu.SMEM((1,), jnp.float32),
            pltpu.SemaphoreType.DMA,
            pltpu.SemaphoreType.DMA,
        ],
    )
    def k(inp_hbm, tgt_hbm, out_hbm, buf0, buf1, cand, tgt_v, loss_v,
          off_ref, zk_ref, sem0, sem1):
        wid = lax.axis_index("s") * 2 + lax.axis_index("c")
        base = wid * b_per_w
        pltpu.sync_copy(tgt_hbm.at[pl.ds(base, b_per_w)], tgt_v)

        pltpu.async_copy(inp_hbm.at[pl.ds(base * ncols, W)], buf0, sem0)
        pltpu.async_copy(inp_hbm.at[pl.ds(base * ncols + W, W)], buf1, sem1)

        def row_body(rl, laccs):
            r = base + rl
            trf = jnp.float32(0.0)
            for w in range(b_per_w // L):
                tw = tgt_v[pl.ds(w * L, L)].astype(jnp.float32)
                trf = trf + jnp.sum(
                    jnp.where(lax.iota(jnp.int32, L) == rl - w * L, tw,
                              jnp.zeros_like(tw)))
            tr = trf.astype(jnp.int32)
            zk_ref[0] = 0.0

            negv = jnp.full((L,), NEG_BIG, jnp.float32)
            zv = jnp.zeros((L,), jnp.float32)
            carry = (negv, (zv,) * NACC, (zv,) * NACC)

            pltpu.make_async_copy(
                inp_hbm.at[pl.ds(r * ncols, W)], buf0, sem0).wait()
            carry = _moments_pass(buf0, 0, tr, carry, zk_ref)
            pltpu.make_async_copy(
                inp_hbm.at[pl.ds(r * ncols + W, W)], buf1, sem1).wait()
            rm, s1t, s2t = _moments_pass(buf1, W, tr, carry, zk_ref)

            m = jnp.max(rm)
            s1 = jnp.sum(s1t[0] + s1t[1] + s1t[2] + s1t[3])
            s2 = jnp.sum(s2t[0] + s2t[1] + s2t[2] + s2t[3])
            thrv = jnp.full((L,), m - 1.0, jnp.float32)

            offv = _filter_pass(buf0, thrv, cand, jnp.zeros((L,), jnp.int32))

            @pl.when(rl < b_per_w - 1)
            def _():
                pltpu.async_copy(
                    inp_hbm.at[pl.ds((r + 1) * ncols, W)], buf0, sem0)

            offv = _filter_pass(buf1, thrv, cand, offv)

            @pl.when(rl < b_per_w - 1)
            def _():
                pltpu.async_copy(
                    inp_hbm.at[pl.ds((r + 1) * ncols + W, W)], buf1, sem1)

            n = jnp.minimum(offv[0], K)

            def mcond(c):
                t_old, t_new, it = c
                return jnp.logical_and(t_old != t_new, it < 32)

            def mbody(c):
                _, t, it = c
                s, cnt = _sum_count_above(cand, n, t)
                tv = jnp.full((L,), s - 1.0, jnp.float32) / jnp.full(
                    (L,), cnt, jnp.float32)
                return t, tv[0], it + 1

            _, tau_x, _ = lax.while_loop(
                mcond, mbody, (jnp.float32(1.0), jnp.float32(NEG_BIG),
                               jnp.int32(0)))

            lane = lax.iota(jnp.int32, L)

            def corr_body(i, c):
                sv, cv = c
                v = cand[pl.ds(i * L, L)]
                valid = (i * L + lane) < n
                msk = jnp.logical_and(v > tau_x, valid)
                z = v - m
                sv = sv + jnp.where(msk, z * z, 0.0)
                cv = cv + jnp.where(msk, 1.0, 0.0)
                return sv, cv

            sv, cv = lax.fori_loop(0, (n + L - 1) // L, corr_body, (zv, zv),
                                   unroll=False)
            corr = jnp.sum(sv)
            cnt = jnp.sum(cv)

            t2 = s2 - 2.0 * m * s1 + jnp.float32(ncols) * m * m
            tau_z = tau_x - m
            s2t_ = t2 - corr - tau_z * tau_z * (jnp.float32(ncols) - cnt)
            lossval = 0.5 * (s2t_ + 1.0) - zk_ref[0]
            return tuple(
                laccs[w] + jnp.where(lane == rl - w * L,
                                     jnp.full((L,), lossval, jnp.float32),
                                     jnp.zeros((L,), jnp.float32))
                for w in range(b_per_w // L))

        laccs = lax.fori_loop(
            0, b_per_w, row_body,
            tuple(jnp.zeros((L,), jnp.float32) for _ in range(b_per_w // L)),
            unroll=False)
        for w in range(b_per_w // L):
            loss_v[pl.ds(w * L, L)] = laccs[w]
        pltpu.sync_copy(loss_v, out_hbm.at[pl.ds(base, b_per_w)])

    return k(inp, target)


def kernel(input, target):
    B, C = input.shape
    assert B % NWORK == 0 and B // NWORK % L == 0
    assert C == 2 * W and W % (VPB * L) == 0 and W % 8 == 0
    losses = _sparsemax_loss_sc(
        input.reshape(-1), target.astype(jnp.int32), b_per_w=B // NWORK,
        ncols=C)
    return jnp.mean(losses)


# speedup vs baseline: 1.9805x; 1.1922x over previous
import functools

import jax
import jax.numpy as jnp
from jax import lax
from jax.experimental import pallas as pl
from jax.experimental.pallas import tpu as pltpu
from jax.experimental.pallas import tpu_sc as plsc

L = 16
NWORK = 32
W = 50000
K = 20480
VPB = 25
NACC = 4
NEG_BIG = -3.0e38


def _select_lane(vec, idx):
    lane = lax.iota(jnp.int32, L)
    return jnp.sum(jnp.where(lane == idx, vec, jnp.zeros_like(vec)))


def _moments_pass(buf, j0, tr, carry, zk_ref):
    in_rng = jnp.logical_and(tr >= j0, tr < j0 + W)
    tloc = jnp.clip(tr - j0, 0, W - 1)

    @pl.when(in_rng)
    def _():
        wbase = pl.multiple_of((tloc // L) * L, L)
        zk_ref[0] = _select_lane(buf[pl.ds(wbase, L)], tloc % L)

    def step(bi, c):
        rm, s1, s2 = c
        base = bi * (VPB * L)
        s1 = list(s1)
        s2 = list(s2)
        bms = []
        for u in range(VPB):
            v = buf[pl.ds(base + u * L, L)]
            s1[u % NACC] = s1[u % NACC] + v
            s2[u % NACC] = s2[u % NACC] + v * v
            if u % NACC == 0:
                bms.append(v)
            else:
                bms[-1] = jnp.maximum(bms[-1], v)
        for b in bms:
            rm = jnp.maximum(rm, b)
        return rm, tuple(s1), tuple(s2)

    return lax.fori_loop(0, W // (VPB * L), step, carry, unroll=False)


def _filter_pass(buf, thrv, cand, offv):
    lanevec = lax.iota(jnp.int32, L)
    lanevec2 = lanevec + L
    kcap = jnp.full((L,), K - 2 * L, jnp.int32)
    stepv = jnp.full((L,), 2 * L, jnp.int32)
    stepv1 = jnp.full((L,), L, jnp.int32)
    zerov = jnp.zeros((L,), jnp.int32)

    def step(bi, offv):
        base = bi * (VPB * L)
        for u in range(0, VPB - 1, 2):
            v0 = buf[pl.ds(base + u * L, L)]
            v1 = buf[pl.ds(base + (u + 1) * L, L)]
            pcnt = plsc.all_reduce_population_count(
                jnp.maximum(v0, v1) >= thrv)
            offc = jnp.minimum(offv, kcap)
            plsc.store_scatter(cand, [offc + lanevec], v0)
            plsc.store_scatter(cand, [offc + lanevec2], v1)
            offv = offc + jnp.where(pcnt >= 1, stepv, zerov)
        vl = buf[pl.ds(base + (VPB - 1) * L, L)]
        pcnt = plsc.all_reduce_population_count(vl >= thrv)
        offc = jnp.minimum(offv, kcap)
        plsc.store_scatter(cand, [offc + lanevec], vl)
        offv = offc + jnp.where(pcnt >= 1, stepv1, zerov)
        return offv

    return lax.fori_loop(0, W // (VPB * L), step, offv, unroll=False)


def _sum_count_above(cand, n, t):
    nv = (n + L - 1) // L
    lane = lax.iota(jnp.int32, L)

    def body(i, c):
        sv, cv = c
        v = cand[pl.ds(i * L, L)]
        valid = (i * L + lane) < n
        msk = jnp.logical_and(v > t, valid)
        sv = sv + jnp.where(msk, v, 0.0)
        cv = cv + jnp.where(msk, 1.0, 0.0)
        return sv, cv

    z = jnp.zeros((L,), jnp.float32)
    sv, cv = lax.fori_loop(0, nv, body, (z, z), unroll=False)
    return jnp.sum(sv), jnp.sum(cv)


def _sparsemax_loss_sc(inp, target, *, b_per_w, ncols):
    mesh = plsc.VectorSubcoreMesh(core_axis_name="c", subcore_axis_name="s")

    @functools.partial(
        pl.kernel,
        out_type=jax.ShapeDtypeStruct((inp.shape[0] // ncols,), jnp.float32),
        mesh=mesh,
        compiler_params=pltpu.CompilerParams(needs_layout_passes=False),
        scratch_types=[
            pltpu.VMEM((W,), jnp.float32),
            pltpu.VMEM((W,), jnp.float32),
            pltpu.VMEM((K,), jnp.float32),
            pltpu.VMEM((b_per_w,), jnp.int32),
            pltpu.VMEM((b_per_w,), jnp.float32),
            pltpu.SMEM((1,), jnp.int32),
            pltpu.SMEM((1,), jnp.float32),
            pltpu.SemaphoreType.DMA,
            pltpu.SemaphoreType.DMA,
        ],
    )
    def k(inp_hbm, tgt_hbm, out_hbm, buf0, buf1, cand, tgt_v, loss_v,
          off_ref, zk_ref, sem0, sem1):
        wid = lax.axis_index("s") * 2 + lax.axis_index("c")
        base = wid * b_per_w
        pltpu.sync_copy(tgt_hbm.at[pl.ds(base, b_per_w)], tgt_v)

        pltpu.async_copy(inp_hbm.at[pl.ds(base * ncols, W)], buf0, sem0)
        pltpu.async_copy(inp_hbm.at[pl.ds(base * ncols + W, W)], buf1, sem1)

        def row_body(rl, laccs):
            r = base + rl
            trf = jnp.float32(0.0)
            for w in range(b_per_w // L):
                tw = tgt_v[pl.ds(w * L, L)].astype(jnp.float32)
                trf = trf + jnp.sum(
                    jnp.where(lax.iota(jnp.int32, L) == rl - w * L, tw,
                              jnp.zeros_like(tw)))
            tr = trf.astype(jnp.int32)
            zk_ref[0] = 0.0

            negv = jnp.full((L,), NEG_BIG, jnp.float32)
            zv = jnp.zeros((L,), jnp.float32)
            carry = (negv, (zv,) * NACC, (zv,) * NACC)

            pltpu.make_async_copy(
                inp_hbm.at[pl.ds(r * ncols, W)], buf0, sem0).wait()
            carry = _moments_pass(buf0, 0, tr, carry, zk_ref)
            pltpu.make_async_copy(
                inp_hbm.at[pl.ds(r * ncols + W, W)], buf1, sem1).wait()
            rm, s1t, s2t = _moments_pass(buf1, W, tr, carry, zk_ref)

            m = jnp.max(rm)
            s1 = jnp.sum(s1t[0] + s1t[1] + s1t[2] + s1t[3])
            s2 = jnp.sum(s2t[0] + s2t[1] + s2t[2] + s2t[3])
            thrv = jnp.full((L,), m - 1.0, jnp.float32)

            offv = _filter_pass(buf0, thrv, cand, jnp.zeros((L,), jnp.int32))

            @pl.when(rl < b_per_w - 1)
            def _():
                pltpu.async_copy(
                    inp_hbm.at[pl.ds((r + 1) * ncols, W)], buf0, sem0)

            offv = _filter_pass(buf1, thrv, cand, offv)

            @pl.when(rl < b_per_w - 1)
            def _():
                pltpu.async_copy(
                    inp_hbm.at[pl.ds((r + 1) * ncols + W, W)], buf1, sem1)

            n = jnp.minimum(offv[0], K)

            def mcond(c):
                t_old, t_new, it = c
                return jnp.logical_and(t_old != t_new, it < 32)

            def mbody(c):
                _, t, it = c
                s, cnt = _sum_count_above(cand, n, t)
                tv = jnp.full((L,), s - 1.0, jnp.float32) / jnp.full(
                    (L,), cnt, jnp.float32)
                return t, tv[0], it + 1

            _, tau_x, _ = lax.while_loop(
                mcond, mbody, (jnp.float32(1.0), jnp.float32(NEG_BIG),
                               jnp.int32(0)))

            lane = lax.iota(jnp.int32, L)

            def corr_body(i, c):
                sv, cv = c
                v = cand[pl.ds(i * L, L)]
                valid = (i * L + lane) < n
                msk = jnp.logical_and(v > tau_x, valid)
                z = v - m
                sv = sv + jnp.where(msk, z * z, 0.0)
                cv = cv + jnp.where(msk, 1.0, 0.0)
                return sv, cv

            sv, cv = lax.fori_loop(0, (n + L - 1) // L, corr_body, (zv, zv),
                                   unroll=False)
            corr = jnp.sum(sv)
            cnt = jnp.sum(cv)

            t2 = s2 - 2.0 * m * s1 + jnp.float32(ncols) * m * m
            tau_z = tau_x - m
            s2t_ = t2 - corr - tau_z * tau_z * (jnp.float32(ncols) - cnt)
            lossval = 0.5 * (s2t_ + 1.0) - zk_ref[0]
            return tuple(
                laccs[w] + jnp.where(lane == rl - w * L,
                                     jnp.full((L,), lossval, jnp.float32),
                                     jnp.zeros((L,), jnp.float32))
                for w in range(b_per_w // L))

        laccs = lax.fori_loop(
            0, b_per_w, row_body,
            tuple(jnp.zeros((L,), jnp.float32) for _ in range(b_per_w // L)),
            unroll=False)
        for w in range(b_per_w // L):
            loss_v[pl.ds(w * L, L)] = laccs[w]
        pltpu.sync_copy(loss_v, out_hbm.at[pl.ds(base, b_per_w)])

    return k(inp, target)


def kernel(input, target):
    B, C = input.shape
    assert B % NWORK == 0 and B // NWORK % L == 0
    assert C == 2 * W and W % (VPB * L) == 0 and W % 8 == 0
    losses = _sparsemax_loss_sc(
        input.reshape(-1), target.astype(jnp.int32), b_per_w=B // NWORK,
        ncols=C)
    return jnp.mean(losses)


# speedup vs baseline: 2.0403x; 1.0302x over previous
import functools

import jax
import jax.numpy as jnp
from jax import lax
from jax.experimental import pallas as pl
from jax.experimental.pallas import tpu as pltpu
from jax.experimental.pallas import tpu_sc as plsc

L = 16
NWORK = 32
W = 50000
K = 20480
VPB = 25
NACC = 4
NEG_BIG = -3.0e38


def _select_lane(vec, idx):
    lane = lax.iota(jnp.int32, L)
    return jnp.sum(jnp.where(lane == idx, vec, jnp.zeros_like(vec)))


def _moments_pass(buf, j0, tr, carry, zk_ref):
    in_rng = jnp.logical_and(tr >= j0, tr < j0 + W)
    tloc = jnp.clip(tr - j0, 0, W - 1)

    @pl.when(in_rng)
    def _():
        wbase = pl.multiple_of((tloc // L) * L, L)
        zk_ref[0] = _select_lane(buf[pl.ds(wbase, L)], tloc % L)

    def step(bi, c):
        rm, s1, s2 = c
        base = bi * (VPB * L)
        s1 = list(s1)
        s2 = list(s2)
        bms = []
        for u in range(VPB):
            v = buf[pl.ds(base + u * L, L)]
            s1[u % NACC] = s1[u % NACC] + v
            s2[u % NACC] = s2[u % NACC] + v * v
            if u % NACC == 0:
                bms.append(v)
            else:
                bms[-1] = jnp.maximum(bms[-1], v)
        for b in bms:
            rm = jnp.maximum(rm, b)
        return rm, tuple(s1), tuple(s2)

    return lax.fori_loop(0, W // (VPB * L), step, carry, unroll=False)


def _filter_pass(buf, thrv, cand, offv):
    lanevec = lax.iota(jnp.int32, L)
    lanevec2 = lanevec + L
    kcap = jnp.full((L,), K - (VPB + 1) * L, jnp.int32)
    stepv = jnp.full((L,), 2 * L, jnp.int32)
    stepv1 = jnp.full((L,), L, jnp.int32)
    zerov = jnp.zeros((L,), jnp.int32)

    def step(bi, offv):
        base = bi * (VPB * L)
        offv = jnp.minimum(offv, kcap)
        for u in range(0, VPB - 1, 2):
            v0 = buf[pl.ds(base + u * L, L)]
            v1 = buf[pl.ds(base + (u + 1) * L, L)]
            pcnt = plsc.all_reduce_population_count(
                jnp.maximum(v0, v1) >= thrv)
            plsc.store_scatter(cand, [offv + lanevec], v0)
            plsc.store_scatter(cand, [offv + lanevec2], v1)
            offv = offv + jnp.where(pcnt >= 1, stepv, zerov)
        vl = buf[pl.ds(base + (VPB - 1) * L, L)]
        pcnt = plsc.all_reduce_population_count(vl >= thrv)
        plsc.store_scatter(cand, [offv + lanevec], vl)
        offv = offv + jnp.where(pcnt >= 1, stepv1, zerov)
        return offv

    return lax.fori_loop(0, W // (VPB * L), step, offv, unroll=False)


def _sum_count_above(cand, n, t):
    nv = (n + L - 1) // L
    lane = lax.iota(jnp.int32, L)

    def body(i, c):
        sv, cv = c
        v = cand[pl.ds(i * L, L)]
        valid = (i * L + lane) < n
        msk = jnp.logical_and(v > t, valid)
        sv = sv + jnp.where(msk, v, 0.0)
        cv = cv + jnp.where(msk, 1.0, 0.0)
        return sv, cv

    z = jnp.zeros((L,), jnp.float32)
    sv, cv = lax.fori_loop(0, nv, body, (z, z), unroll=False)
    return jnp.sum(sv), jnp.sum(cv)


def _sparsemax_loss_sc(inp, target, *, b_per_w, ncols):
    mesh = plsc.VectorSubcoreMesh(core_axis_name="c", subcore_axis_name="s")

    @functools.partial(
        pl.kernel,
        out_type=jax.ShapeDtypeStruct((inp.shape[0] // ncols,), jnp.float32),
        mesh=mesh,
        compiler_params=pltpu.CompilerParams(needs_layout_passes=False),
        scratch_types=[
            pltpu.VMEM((W,), jnp.float32),
            pltpu.VMEM((W,), jnp.float32),
            pltpu.VMEM((K,), jnp.float32),
            pltpu.VMEM((b_per_w,), jnp.int32),
            pltpu.VMEM((b_per_w,), jnp.float32),
            pltpu.SMEM((1,), jnp.int32),
            pltpu.SMEM((1,), jnp.float32),
            pltpu.SemaphoreType.DMA,
            pltpu.SemaphoreType.DMA,
        ],
    )
    def k(inp_hbm, tgt_hbm, out_hbm, buf0, buf1, cand, tgt_v, loss_v,
          off_ref, zk_ref, sem0, sem1):
        wid = lax.axis_index("s") * 2 + lax.axis_index("c")
        base = wid * b_per_w
        pltpu.sync_copy(tgt_hbm.at[pl.ds(base, b_per_w)], tgt_v)

        pltpu.async_copy(inp_hbm.at[pl.ds(base * ncols, W)], buf0, sem0)
        pltpu.async_copy(inp_hbm.at[pl.ds(base * ncols + W, W)], buf1, sem1)

        def row_body(rl, laccs):
            r = base + rl
            trf = jnp.float32(0.0)
            for w in range(b_per_w // L):
                tw = tgt_v[pl.ds(w * L, L)].astype(jnp.float32)
                trf = trf + jnp.sum(
                    jnp.where(lax.iota(jnp.int32, L) == rl - w * L, tw,
                              jnp.zeros_like(tw)))
            tr = trf.astype(jnp.int32)
            zk_ref[0] = 0.0

            negv = jnp.full((L,), NEG_BIG, jnp.float32)
            zv = jnp.zeros((L,), jnp.float32)
            carry = (negv, (zv,) * NACC, (zv,) * NACC)

            pltpu.make_async_copy(
                inp_hbm.at[pl.ds(r * ncols, W)], buf0, sem0).wait()
            carry = _moments_pass(buf0, 0, tr, carry, zk_ref)
            pltpu.make_async_copy(
                inp_hbm.at[pl.ds(r * ncols + W, W)], buf1, sem1).wait()
            rm, s1t, s2t = _moments_pass(buf1, W, tr, carry, zk_ref)

            m = jnp.max(rm)
            s1 = jnp.sum(s1t[0] + s1t[1] + s1t[2] + s1t[3])
            s2 = jnp.sum(s2t[0] + s2t[1] + s2t[2] + s2t[3])
            thrv = jnp.full((L,), m - 1.0, jnp.float32)

            offv = _filter_pass(buf0, thrv, cand, jnp.zeros((L,), jnp.int32))

            @pl.when(rl < b_per_w - 1)
            def _():
                pltpu.async_copy(
                    inp_hbm.at[pl.ds((r + 1) * ncols, W)], buf0, sem0)

            offv = _filter_pass(buf1, thrv, cand, offv)

            @pl.when(rl < b_per_w - 1)
            def _():
                pltpu.async_copy(
                    inp_hbm.at[pl.ds((r + 1) * ncols + W, W)], buf1, sem1)

            n = jnp.minimum(offv[0], K)

            def mcond(c):
                t_old, t_new, it = c
                return jnp.logical_and(t_old != t_new, it < 32)

            def mbody(c):
                _, t, it = c
                s, cnt = _sum_count_above(cand, n, t)
                tv = jnp.full((L,), s - 1.0, jnp.float32) / jnp.full(
                    (L,), cnt, jnp.float32)
                return t, tv[0], it + 1

            _, tau_x, _ = lax.while_loop(
                mcond, mbody, (jnp.float32(1.0), jnp.float32(NEG_BIG),
                               jnp.int32(0)))

            lane = lax.iota(jnp.int32, L)

            def corr_body(i, c):
                sv, cv = c
                v = cand[pl.ds(i * L, L)]
                valid = (i * L + lane) < n
                msk = jnp.logical_and(v > tau_x, valid)
                z = v - m
                sv = sv + jnp.where(msk, z * z, 0.0)
                cv = cv + jnp.where(msk, 1.0, 0.0)
                return sv, cv

            sv, cv = lax.fori_loop(0, (n + L - 1) // L, corr_body, (zv, zv),
                                   unroll=False)
            corr = jnp.sum(sv)
            cnt = jnp.sum(cv)

            t2 = s2 - 2.0 * m * s1 + jnp.float32(ncols) * m * m
            tau_z = tau_x - m
            s2t_ = t2 - corr - tau_z * tau_z * (jnp.float32(ncols) - cnt)
            lossval = 0.5 * (s2t_ + 1.0) - zk_ref[0]
            return tuple(
                laccs[w] + jnp.where(lane == rl - w * L,
                                     jnp.full((L,), lossval, jnp.float32),
                                     jnp.zeros((L,), jnp.float32))
                for w in range(b_per_w // L))

        laccs = lax.fori_loop(
            0, b_per_w, row_body,
            tuple(jnp.zeros((L,), jnp.float32) for _ in range(b_per_w // L)),
            unroll=False)
        for w in range(b_per_w // L):
            loss_v[pl.ds(w * L, L)] = laccs[w]
        pltpu.sync_copy(loss_v, out_hbm.at[pl.ds(base, b_per_w)])

    return k(inp, target)


def kernel(input, target):
    B, C = input.shape
    assert B % NWORK == 0 and B // NWORK % L == 0
    assert C == 2 * W and W % (VPB * L) == 0 and W % 8 == 0
    losses = _sparsemax_loss_sc(
        input.reshape(-1), target.astype(jnp.int32), b_per_w=B // NWORK,
        ncols=C)
    return jnp.mean(losses)
